# baseline (speedup 1.0000x reference)
import functools

import jax
import jax.numpy as jnp
from jax import lax
from jax.experimental import pallas as pl
from jax.experimental.pallas import tpu as pltpu
from jax.experimental.pallas import tpu_sc as plsc

_CAP = 16384
_SEQ = 32
_H = 128
_B = 1024
_K = 8
_D = _SEQ * _H

_QB = 128
_NQB = _B // _QB
_CH = 256
_NCH = _CAP // _CH
_MCH = 2048

_NEG = float(-3.0e38)


_CR = 256
_NCR = _CAP // _CR


def _dense_body(q_ref, e_ref, w1_ref, b1_ref, w2_ref, b2_ref, idx_ref,
                scores_ref, src_ref, hn_ref, st_ref, gmax_ref, amrow_ref):
    i = pl.program_id(0)

    @pl.when(i == 0)
    def _compute_hn():
        w1 = w1_ref[...]
        w2 = w2_ref[...]
        b1 = b1_ref[...]
        b2 = b2_ref[...]
        for c in range(_CAP // _MCH):
            e = e_ref[pl.ds(c * _MCH, _MCH), :]
            h = jnp.maximum(
                jnp.dot(e, w1, preferred_element_type=jnp.float32) + b1, 0.0)
            h = jnp.dot(h, w2, preferred_element_type=jnp.float32) + b2
            nrm = jnp.sqrt(jnp.sum(h * h, axis=1, keepdims=True)) + 1e-8
            hn_ref[pl.ds(c * _MCH, _MCH), :] = h / nrm

    q = q_ref[...]
    qn = q / (jnp.sqrt(jnp.sum(q * q, axis=1, keepdims=True)) + 1e-8)
    for c in range(_NCR):
        hblk = hn_ref[pl.ds(c * _CR, _CR), :]
        stc = lax.dot_general(hblk, qn, (((1,), (1,)), ((), ())),
                              preferred_element_type=jnp.float32)
        st_ref[pl.ds(c * _CR, _CR), :] = stc
        gmax_ref[pl.ds(c, 1), :] = jnp.max(stc, axis=0, keepdims=True)

    ci = lax.broadcasted_iota(jnp.int32, (_NCR, _QB), 0)
    sub8 = lax.broadcasted_iota(jnp.int32, (8, _QB), 0)
    rloc = lax.broadcasted_iota(jnp.int32, (_CR, _QB), 0)
    m_sel = jnp.zeros((8, _QB), jnp.float32)
    am_sel = jnp.zeros((8, _QB), jnp.int32)
    am_hist = []
    for j in range(_K):
        gf = gmax_ref[...]
        m = jnp.max(gf, axis=0, keepdims=True)
        wc = jnp.min(jnp.where(gf == m, ci, _NCR), axis=0, keepdims=True)
        amrow_ref[...] = jnp.full((1, _QB), _CAP, jnp.int32)

        for c in range(_NCR):
            active = wc == c
            off = c * _CR
            blk = st_ref[pl.ds(off, _CR), :]
            excl = None
            for ah in am_hist:
                e = rloc == (ah - off)
                excl = e if excl is None else (excl | e)
            if excl is not None:
                blk = jnp.where(excl, _NEG, blk)
            hit = (blk == m) & active
            bam = jnp.min(jnp.where(hit, rloc, _CAP), axis=0,
                          keepdims=True)
            gmax_ref[pl.ds(c, 1), :] = jnp.max(
                jnp.where(rloc == bam, _NEG, blk), axis=0, keepdims=True)
            amrow_ref[...] = jnp.minimum(
                amrow_ref[...], jnp.where(bam < _CAP, bam + off, _CAP))
        am = amrow_ref[...]
        am_hist.append(am)
        m_sel = jnp.where(sub8 == j, m, m_sel)
        am_sel = jnp.where(sub8 == j, am, am_sel)

    r128 = lax.broadcasted_iota(jnp.int32, (_QB, _QB), 0)
    c128 = lax.broadcasted_iota(jnp.int32, (_QB, _QB), 1)
    eye = (r128 == c128).astype(jnp.float32)
    m_cols = lax.dot_general(eye, m_sel, (((1,), (1,)), ((), ())),
                             precision=lax.Precision.HIGHEST,
                             preferred_element_type=jnp.float32)
    am_cols = lax.dot_general(eye, am_sel.astype(jnp.float32),
                              (((1,), (1,)), ((), ())),
                              precision=lax.Precision.HIGHEST,
                              preferred_element_type=jnp.float32)
    am_cols = (am_cols + 0.5).astype(jnp.int32)

    idx_row = idx_ref[...]
    bcol = lax.broadcasted_iota(jnp.int32, (_QB, _B), 1)
    out_col = lax.broadcasted_iota(jnp.int32, (_QB, 128), 1)
    scores = jnp.zeros((_QB, 128), jnp.float32)
    src = jnp.zeros((_QB, 128), jnp.int32)
    for j in range(_K):
        am_j = am_cols[:, j:j + 1]
        match = idx_row == am_j
        pos = jnp.max(jnp.where(match, bcol, -1), axis=1, keepdims=True)
        sj = jnp.where(pos >= 0, _CAP + pos, am_j)
        scores = jnp.where(out_col == j, m_cols[:, j:j + 1], scores)
        src = jnp.where(out_col == j, sj, src)
    scores_ref[...] = scores
    src_ref[...] = src


def _dense_call(query, emb, W1, b1, W2, b2, idx, interpret=False):
    return pl.pallas_call(
        _dense_body,
        grid=(_NQB,),
        in_specs=[
            pl.BlockSpec((_QB, _H), lambda i: (i, 0)),
            pl.BlockSpec((_CAP, _H), lambda i: (0, 0)),
            pl.BlockSpec((_H, _H), lambda i: (0, 0)),
            pl.BlockSpec((1, _H), lambda i: (0, 0)),
            pl.BlockSpec((_H, _H), lambda i: (0, 0)),
            pl.BlockSpec((1, _H), lambda i: (0, 0)),
            pl.BlockSpec((1, _B), lambda i: (0, 0)),
        ],
        out_specs=[
            pl.BlockSpec((_QB, 128), lambda i: (i, 0)),
            pl.BlockSpec((_QB, 128), lambda i: (i, 0)),
        ],
        out_shape=[
            jax.ShapeDtypeStruct((_B, 128), jnp.float32),
            jax.ShapeDtypeStruct((_B, 128), jnp.int32),
        ],
        scratch_shapes=[
            pltpu.VMEM((_CAP, _H), jnp.float32),
            pltpu.VMEM((_CAP, _QB), jnp.float32),
            pltpu.VMEM((_NCR, _QB), jnp.float32),
            pltpu.VMEM((1, _QB), jnp.int32),
        ],
        interpret=interpret,
    )(query, emb, W1, b1.reshape(1, _H), W2, b2.reshape(1, _H),
      idx.reshape(1, _B))


_NW = 32
_EW = (_B * _K) // _NW
_GB = 8


def _sc_body(src_hbm, ep_hbm, val_hbm, out_hbm,
             src_v, eidx_v, bufa, bufb, vbuf, sema, semb):
    c_id = lax.axis_index("c")
    s_id = lax.axis_index("s")
    wid = s_id * 2 + c_id
    base = wid * _EW
    pltpu.sync_copy(src_hbm.at[pl.ds(base, _EW)], src_v)
    for i in range(_EW // 16):
        v = src_v[pl.ds(i * 16, 16)]
        eidx_v[pl.ds(i * 16, 16)] = jnp.where(v >= _CAP, 0, v)

    bufs = (bufa, bufb)
    sems = (sema, semb)
    nch = _EW // _GB
    pend = pltpu.async_copy(ep_hbm.at[eidx_v.at[pl.ds(0, _GB)]],
                            bufs[0], sems[0])
    for c in range(nch):
        nxt = None
        if c + 1 < nch:
            nxt = pltpu.async_copy(
                ep_hbm.at[eidx_v.at[pl.ds((c + 1) * _GB, _GB)]],
                bufs[(c + 1) % 2], sems[(c + 1) % 2])
        pend.wait()
        pltpu.sync_copy(bufs[c % 2], out_hbm.at[pl.ds(base + c * _GB, _GB)])
        pend = nxt

    def p2(g, carry):
        v = src_v[pl.ds(g * 16, 16)]
        for l in range(16):
            s = v[l]

            @pl.when(s >= _CAP)
            def _(s=s, l=l):
                pltpu.sync_copy(val_hbm.at[pl.ds(s - _CAP, 1)], vbuf)
                pltpu.sync_copy(vbuf,
                                out_hbm.at[pl.ds(base + g * 16 + l, 1)])

        return carry

    lax.fori_loop(0, _EW // 16, p2, 0)


def _sc_gather(src, episodes, val):
    mesh = plsc.VectorSubcoreMesh(core_axis_name="c", subcore_axis_name="s")
    f = functools.partial(
        pl.kernel,
        mesh=mesh,
        out_type=jax.ShapeDtypeStruct((_B * _K, _SEQ, _H), jnp.float32),
        scratch_types=[
            pltpu.VMEM((_EW,), jnp.int32),
            pltpu.VMEM((_EW,), jnp.int32),
            pltpu.VMEM((_GB, _SEQ, _H), jnp.float32),
            pltpu.VMEM((_GB, _SEQ, _H), jnp.float32),
            pltpu.VMEM((1, _SEQ, _H), jnp.float32),
            pltpu.SemaphoreType.DMA,
            pltpu.SemaphoreType.DMA,
        ],
    )(_sc_body)
    return f(src, episodes, val)


def kernel(query, val, idx, episodes, episode_embeddings, W1, b1, W2, b2, k):
    scores_pad, src_pad = _dense_call(query, episode_embeddings,
                                      W1, b1, W2, b2, idx)
    top_scores = scores_pad[:, :_K]
    src = src_pad[:, :_K].reshape(_B * _K)
    out = _sc_gather(src, episodes, val)
    retrieved = out.reshape(_B, _K, _SEQ, _H)
    return retrieved, top_scores


# speedup vs baseline: 1.6397x; 1.6397x over previous
import functools

import jax
import jax.numpy as jnp
from jax import lax
from jax.experimental import pallas as pl
from jax.experimental.pallas import tpu as pltpu
from jax.experimental.pallas import tpu_sc as plsc

_CAP = 16384
_SEQ = 32
_H = 128
_B = 1024
_K = 8
_D = _SEQ * _H

_QB = 128
_NQB = _B // _QB
_CH = 256
_NCH = _CAP // _CH
_MCH = 2048

_NEG = float(-3.0e38)


_CR = 256
_NCR = _CAP // _CR


def _dense_body(q_ref, e_ref, w1_ref, b1_ref, w2_ref, b2_ref, idx_ref,
                scores_ref, src_ref, hn_ref, st_ref, gmax_ref, amrow_ref):
    i = pl.program_id(0)

    @pl.when(i == 0)
    def _compute_hn():
        w1 = w1_ref[...]
        w2 = w2_ref[...]
        b1 = b1_ref[...]
        b2 = b2_ref[...]
        for c in range(_CAP // _MCH):
            e = e_ref[pl.ds(c * _MCH, _MCH), :]
            h = jnp.maximum(
                jnp.dot(e, w1, preferred_element_type=jnp.float32) + b1, 0.0)
            h = jnp.dot(h, w2, preferred_element_type=jnp.float32) + b2
            nrm = jnp.sqrt(jnp.sum(h * h, axis=1, keepdims=True)) + 1e-8
            hn_ref[pl.ds(c * _MCH, _MCH), :] = h / nrm

    q = q_ref[...]
    qn = q / (jnp.sqrt(jnp.sum(q * q, axis=1, keepdims=True)) + 1e-8)
    for c in range(_NCR):
        hblk = hn_ref[pl.ds(c * _CR, _CR), :]
        stc = lax.dot_general(hblk, qn, (((1,), (1,)), ((), ())),
                              preferred_element_type=jnp.float32)
        st_ref[pl.ds(c * _CR, _CR), :] = stc
        gmax_ref[pl.ds(c, 1), :] = jnp.max(stc, axis=0, keepdims=True)

    ci = lax.broadcasted_iota(jnp.int32, (_NCR, _QB), 0)
    sub8 = lax.broadcasted_iota(jnp.int32, (8, _QB), 0)
    rloc = lax.broadcasted_iota(jnp.int32, (_CR, _QB), 0)
    m_sel = jnp.zeros((8, _QB), jnp.float32)
    am_sel = jnp.zeros((8, _QB), jnp.int32)
    for j in range(_K):
        gf = gmax_ref[...]
        m = jnp.max(gf, axis=0, keepdims=True)
        wc = jnp.min(jnp.where(gf == m, ci, _NCR), axis=0, keepdims=True)
        amrow_ref[...] = jnp.full((1, _QB), _CAP, jnp.int32)

        for c in range(_NCR):
            active = wc == c
            off = c * _CR
            blk = st_ref[pl.ds(off, _CR), :]
            hit = (blk == m) & active
            bam = jnp.min(jnp.where(hit, rloc, _CAP), axis=0,
                          keepdims=True)
            blk2 = jnp.where(rloc == bam, _NEG, blk)
            st_ref[pl.ds(off, _CR), :] = blk2
            gmax_ref[pl.ds(c, 1), :] = jnp.max(blk2, axis=0, keepdims=True)
            amrow_ref[...] = jnp.minimum(
                amrow_ref[...], jnp.where(bam < _CAP, bam + off, _CAP))
        am = amrow_ref[...]
        m_sel = jnp.where(sub8 == j, m, m_sel)
        am_sel = jnp.where(sub8 == j, am, am_sel)

    r128 = lax.broadcasted_iota(jnp.int32, (_QB, _QB), 0)
    c128 = lax.broadcasted_iota(jnp.int32, (_QB, _QB), 1)
    eye = (r128 == c128).astype(jnp.float32)
    m_cols = lax.dot_general(eye, m_sel, (((1,), (1,)), ((), ())),
                             precision=lax.Precision.HIGHEST,
                             preferred_element_type=jnp.float32)
    am_cols = lax.dot_general(eye, am_sel.astype(jnp.float32),
                              (((1,), (1,)), ((), ())),
                              precision=lax.Precision.HIGHEST,
                              preferred_element_type=jnp.float32)
    am_cols = (am_cols + 0.5).astype(jnp.int32)

    idx_row = idx_ref[...]
    bcol = lax.broadcasted_iota(jnp.int32, (_QB, _B), 1)
    out_col = lax.broadcasted_iota(jnp.int32, (_QB, 128), 1)
    scores = jnp.zeros((_QB, 128), jnp.float32)
    src = jnp.zeros((_QB, 128), jnp.int32)
    for j in range(_K):
        am_j = am_cols[:, j:j + 1]
        match = idx_row == am_j
        pos = jnp.max(jnp.where(match, bcol, -1), axis=1, keepdims=True)
        sj = jnp.where(pos >= 0, _CAP + pos, am_j)
        scores = jnp.where(out_col == j, m_cols[:, j:j + 1], scores)
        src = jnp.where(out_col == j, sj, src)
    scores_ref[...] = scores
    src_ref[...] = src


def _dense_call(query, emb, W1, b1, W2, b2, idx, interpret=False):
    return pl.pallas_call(
        _dense_body,
        grid=(_NQB,),
        in_specs=[
            pl.BlockSpec((_QB, _H), lambda i: (i, 0)),
            pl.BlockSpec((_CAP, _H), lambda i: (0, 0)),
            pl.BlockSpec((_H, _H), lambda i: (0, 0)),
            pl.BlockSpec((1, _H), lambda i: (0, 0)),
            pl.BlockSpec((_H, _H), lambda i: (0, 0)),
            pl.BlockSpec((1, _H), lambda i: (0, 0)),
            pl.BlockSpec((1, _B), lambda i: (0, 0)),
        ],
        out_specs=[
            pl.BlockSpec((_QB, 128), lambda i: (i, 0)),
            pl.BlockSpec((_QB, 128), lambda i: (i, 0)),
        ],
        out_shape=[
            jax.ShapeDtypeStruct((_B, 128), jnp.float32),
            jax.ShapeDtypeStruct((_B, 128), jnp.int32),
        ],
        scratch_shapes=[
            pltpu.VMEM((_CAP, _H), jnp.float32),
            pltpu.VMEM((_CAP, _QB), jnp.float32),
            pltpu.VMEM((_NCR, _QB), jnp.float32),
            pltpu.VMEM((1, _QB), jnp.int32),
        ],
        interpret=interpret,
    )(query, emb, W1, b1.reshape(1, _H), W2, b2.reshape(1, _H),
      idx.reshape(1, _B))


_NW = 32
_EW = (_B * _K) // _NW
_GB = 8


def _sc_body(src_hbm, ep_hbm, val_hbm, out_hbm,
             src_v, eidx_v, bufa, bufb, vbuf, sema, semb):
    c_id = lax.axis_index("c")
    s_id = lax.axis_index("s")
    wid = s_id * 2 + c_id
    base = wid * _EW
    pltpu.sync_copy(src_hbm.at[pl.ds(base, _EW)], src_v)
    for i in range(_EW // 16):
        v = src_v[pl.ds(i * 16, 16)]
        eidx_v[pl.ds(i * 16, 16)] = jnp.where(v >= _CAP, 0, v)

    bufs = (bufa, bufb)
    sems = (sema, semb)
    nch = _EW // _GB
    pend = pltpu.async_copy(ep_hbm.at[eidx_v.at[pl.ds(0, _GB)]],
                            bufs[0], sems[0])
    for c in range(nch):
        nxt = None
        if c + 1 < nch:
            nxt = pltpu.async_copy(
                ep_hbm.at[eidx_v.at[pl.ds((c + 1) * _GB, _GB)]],
                bufs[(c + 1) % 2], sems[(c + 1) % 2])
        pend.wait()
        pltpu.sync_copy(bufs[c % 2], out_hbm.at[pl.ds(base + c * _GB, _GB)])
        pend = nxt

    def p2(g, carry):
        v = src_v[pl.ds(g * 16, 16)]
        for l in range(16):
            s = v[l]

            @pl.when(s >= _CAP)
            def _(s=s, l=l):
                pltpu.sync_copy(val_hbm.at[pl.ds(s - _CAP, 1)], vbuf)
                pltpu.sync_copy(vbuf,
                                out_hbm.at[pl.ds(base + g * 16 + l, 1)])

        return carry

    lax.fori_loop(0, _EW // 16, p2, 0)


def _sc_gather(src, episodes, val):
    mesh = plsc.VectorSubcoreMesh(core_axis_name="c", subcore_axis_name="s")
    f = functools.partial(
        pl.kernel,
        mesh=mesh,
        out_type=jax.ShapeDtypeStruct((_B * _K, _SEQ, _H), jnp.float32),
        scratch_types=[
            pltpu.VMEM((_EW,), jnp.int32),
            pltpu.VMEM((_EW,), jnp.int32),
            pltpu.VMEM((_GB, _SEQ, _H), jnp.float32),
            pltpu.VMEM((_GB, _SEQ, _H), jnp.float32),
            pltpu.VMEM((1, _SEQ, _H), jnp.float32),
            pltpu.SemaphoreType.DMA,
            pltpu.SemaphoreType.DMA,
        ],
    )(_sc_body)
    return f(src, episodes, val)


def kernel(query, val, idx, episodes, episode_embeddings, W1, b1, W2, b2, k):
    scores_pad, src_pad = _dense_call(query, episode_embeddings,
                                      W1, b1, W2, b2, idx)
    top_scores = scores_pad[:, :_K]
    src = src_pad[:, :_K].reshape(_B * _K)
    out = _sc_gather(src, episodes, val)
    retrieved = out.reshape(_B, _K, _SEQ, _H)
    return retrieved, top_scores


# speedup vs baseline: 1.6754x; 1.0218x over previous
import functools

import jax
import jax.numpy as jnp
from jax import lax
from jax.experimental import pallas as pl
from jax.experimental.pallas import tpu as pltpu
from jax.experimental.pallas import tpu_sc as plsc

_CAP = 16384
_SEQ = 32
_H = 128
_B = 1024
_K = 8
_D = _SEQ * _H

_QB = 128
_NQB = _B // _QB
_CH = 256
_NCH = _CAP // _CH
_MCH = 2048

_NEG = float(-3.0e38)


_CR = 256
_NCR = _CAP // _CR


def _dense_body(q_ref, e_ref, w1_ref, b1_ref, w2_ref, b2_ref, idx_ref,
                scores_ref, src_ref, hn_ref, st_ref, gmax_ref, amrow_ref):
    i = pl.program_id(0)

    @pl.when(i == 0)
    def _compute_hn():
        w1 = w1_ref[...]
        w2 = w2_ref[...]
        b1 = b1_ref[...]
        b2 = b2_ref[...]
        for c in range(_CAP // _MCH):
            e = e_ref[pl.ds(c * _MCH, _MCH), :]
            h = jnp.maximum(
                jnp.dot(e, w1, preferred_element_type=jnp.float32) + b1, 0.0)
            h = jnp.dot(h, w2, preferred_element_type=jnp.float32) + b2
            nrm = jnp.sqrt(jnp.sum(h * h, axis=1, keepdims=True)) + 1e-8
            hn_ref[pl.ds(c * _MCH, _MCH), :] = h / nrm

    q = q_ref[...]
    qn = q / (jnp.sqrt(jnp.sum(q * q, axis=1, keepdims=True)) + 1e-8)
    for c in range(_NCR):
        hblk = hn_ref[pl.ds(c * _CR, _CR), :]
        stc = lax.dot_general(hblk, qn, (((1,), (1,)), ((), ())),
                              preferred_element_type=jnp.float32)
        st_ref[pl.ds(c * _CR, _CR), :] = stc
        gmax_ref[pl.ds(c, 1), :] = jnp.max(stc, axis=0, keepdims=True)

    ci = lax.broadcasted_iota(jnp.int32, (_NCR, _QB), 0)
    sub8 = lax.broadcasted_iota(jnp.int32, (8, _QB), 0)
    rloc = lax.broadcasted_iota(jnp.int32, (_CR, _QB), 0)
    m_sel = jnp.zeros((8, _QB), jnp.float32)
    am_sel = jnp.zeros((8, _QB), jnp.int32)
    for j in range(_K):
        gf = gmax_ref[...]
        m = jnp.max(gf, axis=0, keepdims=True)
        wc = jnp.min(jnp.where(gf == m, ci, _NCR), axis=0, keepdims=True)
        am = jnp.full((1, _QB), _CAP, jnp.int32)

        for c in range(_NCR):
            active = wc == c
            off = c * _CR
            blk = st_ref[pl.ds(off, _CR), :]
            hit = (blk == m) & active
            bam = jnp.min(jnp.where(hit, rloc, _CAP), axis=0,
                          keepdims=True)
            if j < _K - 1:
                blk2 = jnp.where(rloc == bam, _NEG, blk)
                st_ref[pl.ds(off, _CR), :] = blk2
                gmax_ref[pl.ds(c, 1), :] = jnp.max(blk2, axis=0,
                                                   keepdims=True)
            am = jnp.minimum(am, jnp.where(bam < _CAP, bam + off, _CAP))
        m_sel = jnp.where(sub8 == j, m, m_sel)
        am_sel = jnp.where(sub8 == j, am, am_sel)

    r128 = lax.broadcasted_iota(jnp.int32, (_QB, _QB), 0)
    c128 = lax.broadcasted_iota(jnp.int32, (_QB, _QB), 1)
    eye = (r128 == c128).astype(jnp.float32)
    m_cols = lax.dot_general(eye, m_sel, (((1,), (1,)), ((), ())),
                             precision=lax.Precision.HIGHEST,
                             preferred_element_type=jnp.float32)
    am_cols = lax.dot_general(eye, am_sel.astype(jnp.float32),
                              (((1,), (1,)), ((), ())),
                              precision=lax.Precision.HIGHEST,
                              preferred_element_type=jnp.float32)
    am_cols = (am_cols + 0.5).astype(jnp.int32)

    idx_row = idx_ref[...]
    bcol = lax.broadcasted_iota(jnp.int32, (_QB, _B), 1)
    out_col = lax.broadcasted_iota(jnp.int32, (_QB, 128), 1)
    scores = jnp.zeros((_QB, 128), jnp.float32)
    src = jnp.zeros((_QB, 128), jnp.int32)
    for j in range(_K):
        am_j = am_cols[:, j:j + 1]
        match = idx_row == am_j
        pos = jnp.max(jnp.where(match, bcol, -1), axis=1, keepdims=True)
        sj = jnp.where(pos >= 0, _CAP + pos, am_j)
        scores = jnp.where(out_col == j, m_cols[:, j:j + 1], scores)
        src = jnp.where(out_col == j, sj, src)
    scores_ref[...] = scores
    src_ref[...] = src


def _dense_call(query, emb, W1, b1, W2, b2, idx, interpret=False):
    return pl.pallas_call(
        _dense_body,
        grid=(_NQB,),
        in_specs=[
            pl.BlockSpec((_QB, _H), lambda i: (i, 0)),
            pl.BlockSpec((_CAP, _H), lambda i: (0, 0)),
            pl.BlockSpec((_H, _H), lambda i: (0, 0)),
            pl.BlockSpec((1, _H), lambda i: (0, 0)),
            pl.BlockSpec((_H, _H), lambda i: (0, 0)),
            pl.BlockSpec((1, _H), lambda i: (0, 0)),
            pl.BlockSpec((1, _B), lambda i: (0, 0)),
        ],
        out_specs=[
            pl.BlockSpec((_QB, 128), lambda i: (i, 0)),
            pl.BlockSpec((_QB, 128), lambda i: (i, 0)),
        ],
        out_shape=[
            jax.ShapeDtypeStruct((_B, 128), jnp.float32),
            jax.ShapeDtypeStruct((_B, 128), jnp.int32),
        ],
        scratch_shapes=[
            pltpu.VMEM((_CAP, _H), jnp.float32),
            pltpu.VMEM((_CAP, _QB), jnp.float32),
            pltpu.VMEM((_NCR, _QB), jnp.float32),
            pltpu.VMEM((1, _QB), jnp.int32),
        ],
        interpret=interpret,
    )(query, emb, W1, b1.reshape(1, _H), W2, b2.reshape(1, _H),
      idx.reshape(1, _B))


_NW = 32
_EW = (_B * _K) // _NW
_GB = 8


def _sc_body(src_hbm, ep_hbm, val_hbm, out_hbm,
             src_v, eidx_v, bufa, bufb, vbuf, sema, semb):
    c_id = lax.axis_index("c")
    s_id = lax.axis_index("s")
    wid = s_id * 2 + c_id
    base = wid * _EW
    pltpu.sync_copy(src_hbm.at[pl.ds(base, _EW)], src_v)
    for i in range(_EW // 16):
        v = src_v[pl.ds(i * 16, 16)]
        eidx_v[pl.ds(i * 16, 16)] = jnp.where(v >= _CAP, 0, v)

    bufs = (bufa, bufb)
    sems = (sema, semb)
    nch = _EW // _GB
    pend = pltpu.async_copy(ep_hbm.at[eidx_v.at[pl.ds(0, _GB)]],
                            bufs[0], sems[0])
    for c in range(nch):
        nxt = None
        if c + 1 < nch:
            nxt = pltpu.async_copy(
                ep_hbm.at[eidx_v.at[pl.ds((c + 1) * _GB, _GB)]],
                bufs[(c + 1) % 2], sems[(c + 1) % 2])
        pend.wait()
        pltpu.sync_copy(bufs[c % 2], out_hbm.at[pl.ds(base + c * _GB, _GB)])
        pend = nxt

    def p2(g, carry):
        v = src_v[pl.ds(g * 16, 16)]
        for l in range(16):
            s = v[l]

            @pl.when(s >= _CAP)
            def _(s=s, l=l):
                pltpu.sync_copy(val_hbm.at[pl.ds(s - _CAP, 1)], vbuf)
                pltpu.sync_copy(vbuf,
                                out_hbm.at[pl.ds(base + g * 16 + l, 1)])

        return carry

    lax.fori_loop(0, _EW // 16, p2, 0)


def _sc_gather(src, episodes, val):
    mesh = plsc.VectorSubcoreMesh(core_axis_name="c", subcore_axis_name="s")
    f = functools.partial(
        pl.kernel,
        mesh=mesh,
        out_type=jax.ShapeDtypeStruct((_B * _K, _SEQ, _H), jnp.float32),
        scratch_types=[
            pltpu.VMEM((_EW,), jnp.int32),
            pltpu.VMEM((_EW,), jnp.int32),
            pltpu.VMEM((_GB, _SEQ, _H), jnp.float32),
            pltpu.VMEM((_GB, _SEQ, _H), jnp.float32),
            pltpu.VMEM((1, _SEQ, _H), jnp.float32),
            pltpu.SemaphoreType.DMA,
            pltpu.SemaphoreType.DMA,
        ],
    )(_sc_body)
    return f(src, episodes, val)


def kernel(query, val, idx, episodes, episode_embeddings, W1, b1, W2, b2, k):
    scores_pad, src_pad = _dense_call(query, episode_embeddings,
                                      W1, b1, W2, b2, idx)
    top_scores = scores_pad[:, :_K]
    src = src_pad[:, :_K].reshape(_B * _K)
    out = _sc_gather(src, episodes, val)
    retrieved = out.reshape(_B, _K, _SEQ, _H)
    return retrieved, top_scores


# speedup vs baseline: 1.6763x; 1.0006x over previous
import functools

import jax
import jax.numpy as jnp
from jax import lax
from jax.experimental import pallas as pl
from jax.experimental.pallas import tpu as pltpu
from jax.experimental.pallas import tpu_sc as plsc

_CAP = 16384
_SEQ = 32
_H = 128
_B = 1024
_K = 8
_D = _SEQ * _H

_QB = 128
_NQB = _B // _QB
_CH = 256
_NCH = _CAP // _CH
_MCH = 2048

_NEG = float(-3.0e38)


_CR = 256
_NCR = _CAP // _CR


def _dense_body(q_ref, e_ref, w1_ref, b1_ref, w2_ref, b2_ref, idx_ref,
                scores_ref, src_ref, hn_ref, st_ref, gmax_ref):
    i = pl.program_id(0)

    @pl.when(i == 0)
    def _compute_hn():
        w1 = w1_ref[...]
        w2 = w2_ref[...]
        b1 = b1_ref[...]
        b2 = b2_ref[...]
        for c in range(_CAP // _MCH):
            e = e_ref[pl.ds(c * _MCH, _MCH), :]
            h = jnp.maximum(
                jnp.dot(e, w1, preferred_element_type=jnp.float32) + b1, 0.0)
            h = jnp.dot(h, w2, preferred_element_type=jnp.float32) + b2
            nrm = jnp.sqrt(jnp.sum(h * h, axis=1, keepdims=True)) + 1e-8
            hn_ref[pl.ds(c * _MCH, _MCH), :] = h / nrm

    q = q_ref[...]
    qn = q / (jnp.sqrt(jnp.sum(q * q, axis=1, keepdims=True)) + 1e-8)
    for c in range(_NCR):
        hblk = hn_ref[pl.ds(c * _CR, _CR), :]
        stc = lax.dot_general(hblk, qn, (((1,), (1,)), ((), ())),
                              preferred_element_type=jnp.float32)
        st_ref[pl.ds(c * _CR, _CR), :] = stc
        gmax_ref[pl.ds(c, 1), :] = jnp.max(stc, axis=0, keepdims=True)

    ci = lax.broadcasted_iota(jnp.int32, (_NCR, _QB), 0)
    sub8 = lax.broadcasted_iota(jnp.int32, (8, _QB), 0)
    rloc = lax.broadcasted_iota(jnp.int32, (_CR, _QB), 0)
    m_sel = jnp.zeros((8, _QB), jnp.float32)
    am_sel = jnp.zeros((8, _QB), jnp.int32)
    for j in range(_K):
        gf = gmax_ref[...]
        m = jnp.max(gf, axis=0, keepdims=True)
        wc = jnp.min(jnp.where(gf == m, ci, _NCR), axis=0, keepdims=True)
        am = jnp.full((1, _QB), _CAP, jnp.int32)

        for c in range(_NCR):
            active = wc == c
            off = c * _CR
            blk = st_ref[pl.ds(off, _CR), :]
            hit = (blk == m) & active
            bam = jnp.min(jnp.where(hit, rloc, _CAP), axis=0,
                          keepdims=True)
            if j < _K - 1:
                blk2 = jnp.where(rloc == bam, _NEG, blk)
                st_ref[pl.ds(off, _CR), :] = blk2
                gmax_ref[pl.ds(c, 1), :] = jnp.max(blk2, axis=0,
                                                   keepdims=True)
            am = jnp.minimum(am, jnp.where(bam < _CAP, bam + off, _CAP))
        m_sel = jnp.where(sub8 == j, m, m_sel)
        am_sel = jnp.where(sub8 == j, am, am_sel)

    r128 = lax.broadcasted_iota(jnp.int32, (_QB, _QB), 0)
    c128 = lax.broadcasted_iota(jnp.int32, (_QB, _QB), 1)
    eye = (r128 == c128).astype(jnp.float32)
    m_cols = lax.dot_general(eye, m_sel, (((1,), (1,)), ((), ())),
                             precision=lax.Precision.HIGHEST,
                             preferred_element_type=jnp.float32)
    am_cols = lax.dot_general(eye, am_sel.astype(jnp.float32),
                              (((1,), (1,)), ((), ())),
                              precision=lax.Precision.HIGHEST,
                              preferred_element_type=jnp.float32)
    am_cols = (am_cols + 0.5).astype(jnp.int32)

    idx_row = idx_ref[...]
    bcol = lax.broadcasted_iota(jnp.int32, (_QB, _B), 1)
    out_col = lax.broadcasted_iota(jnp.int32, (_QB, 128), 1)
    scores = jnp.zeros((_QB, 128), jnp.float32)
    src = jnp.zeros((_QB, 128), jnp.int32)
    for j in range(_K):
        am_j = am_cols[:, j:j + 1]
        match = idx_row == am_j
        pos = jnp.max(jnp.where(match, bcol, -1), axis=1, keepdims=True)
        sj = jnp.where(pos >= 0, _CAP + pos, am_j)
        scores = jnp.where(out_col == j, m_cols[:, j:j + 1], scores)
        src = jnp.where(out_col == j, sj, src)
    scores_ref[...] = scores
    src_ref[...] = src


def _dense_call(query, emb, W1, b1, W2, b2, idx, interpret=False):
    return pl.pallas_call(
        _dense_body,
        grid=(_NQB,),
        in_specs=[
            pl.BlockSpec((_QB, _H), lambda i: (i, 0)),
            pl.BlockSpec((_CAP, _H), lambda i: (0, 0)),
            pl.BlockSpec((_H, _H), lambda i: (0, 0)),
            pl.BlockSpec((1, _H), lambda i: (0, 0)),
            pl.BlockSpec((_H, _H), lambda i: (0, 0)),
            pl.BlockSpec((1, _H), lambda i: (0, 0)),
            pl.BlockSpec((1, _B), lambda i: (0, 0)),
        ],
        out_specs=[
            pl.BlockSpec((_QB, 128), lambda i: (i, 0)),
            pl.BlockSpec((_QB, 128), lambda i: (i, 0)),
        ],
        out_shape=[
            jax.ShapeDtypeStruct((_B, 128), jnp.float32),
            jax.ShapeDtypeStruct((_B, 128), jnp.int32),
        ],
        scratch_shapes=[
            pltpu.VMEM((_CAP, _H), jnp.float32),
            pltpu.VMEM((_CAP, _QB), jnp.float32),
            pltpu.VMEM((_NCR, _QB), jnp.float32),
        ],
        interpret=interpret,
    )(query, emb, W1, b1.reshape(1, _H), W2, b2.reshape(1, _H),
      idx.reshape(1, _B))


_NW = 32
_EW = (_B * _K) // _NW
_GB = 8


def _sc_body(src_hbm, ep_hbm, val_hbm, out_hbm,
             src_v, eidx_v, bufa, bufb, bufc, vbuf,
             sema, semb, semc, wsema, wsemb, wsemc):
    c_id = lax.axis_index("c")
    s_id = lax.axis_index("s")
    wid = s_id * 2 + c_id
    base = wid * _EW
    pltpu.sync_copy(src_hbm.at[pl.ds(base, _EW)], src_v)
    for i in range(_EW // 16):
        v = src_v[pl.ds(i * 16, 16)]
        eidx_v[pl.ds(i * 16, 16)] = jnp.where(v >= _CAP, 0, v)

    bufs = (bufa, bufb, bufc)
    gsems = (sema, semb, semc)
    wsems = (wsema, wsemb, wsemc)
    nch = _EW // _GB
    nbuf = 3
    gp = [None] * nbuf
    wp = [None] * nbuf
    for c in range(min(nbuf, nch)):
        gp[c] = pltpu.async_copy(ep_hbm.at[eidx_v.at[pl.ds(c * _GB, _GB)]],
                                 bufs[c], gsems[c])
    for c in range(nch):
        s = c % nbuf
        gp[s].wait()
        wp[s] = pltpu.async_copy(bufs[s],
                                 out_hbm.at[pl.ds(base + c * _GB, _GB)],
                                 wsems[s])
        n = c + nbuf
        if n < nch:
            wp[s].wait()
            gp[s] = pltpu.async_copy(
                ep_hbm.at[eidx_v.at[pl.ds(n * _GB, _GB)]], bufs[s], gsems[s])
    for c in range(max(0, nch - nbuf), nch):
        wp[c % nbuf].wait()

    def p2(g, carry):
        v = src_v[pl.ds(g * 16, 16)]
        for l in range(16):
            s = v[l]

            @pl.when(s >= _CAP)
            def _(s=s, l=l):
                pltpu.sync_copy(val_hbm.at[pl.ds(s - _CAP, 1)], vbuf)
                pltpu.sync_copy(vbuf,
                                out_hbm.at[pl.ds(base + g * 16 + l, 1)])

        return carry

    lax.fori_loop(0, _EW // 16, p2, 0)


def _sc_gather(src, episodes, val):
    mesh = plsc.VectorSubcoreMesh(core_axis_name="c", subcore_axis_name="s")
    f = functools.partial(
        pl.kernel,
        mesh=mesh,
        out_type=jax.ShapeDtypeStruct((_B * _K, _SEQ, _H), jnp.float32),
        scratch_types=[
            pltpu.VMEM((_EW,), jnp.int32),
            pltpu.VMEM((_EW,), jnp.int32),
            pltpu.VMEM((_GB, _SEQ, _H), jnp.float32),
            pltpu.VMEM((_GB, _SEQ, _H), jnp.float32),
            pltpu.VMEM((_GB, _SEQ, _H), jnp.float32),
            pltpu.VMEM((1, _SEQ, _H), jnp.float32),
            pltpu.SemaphoreType.DMA,
            pltpu.SemaphoreType.DMA,
            pltpu.SemaphoreType.DMA,
            pltpu.SemaphoreType.DMA,
            pltpu.SemaphoreType.DMA,
            pltpu.SemaphoreType.DMA,
        ],
    )(_sc_body)
    return f(src, episodes, val)


def kernel(query, val, idx, episodes, episode_embeddings, W1, b1, W2, b2, k):
    scores_pad, src_pad = _dense_call(query, episode_embeddings,
                                      W1, b1, W2, b2, idx)
    top_scores = scores_pad[:, :_K]
    src = src_pad[:, :_K].reshape(_B * _K)
    out = _sc_gather(src, episodes, val)
    retrieved = out.reshape(_B, _K, _SEQ, _H)
    return retrieved, top_scores
